# baseline (speedup 1.0000x reference)
import functools

import jax
import jax.numpy as jnp
from jax import lax
from jax.experimental import pallas as pl
from jax.experimental.pallas import tpu as pltpu
from jax.experimental.pallas import tpu_sc as plsc

N_NODES = 50000
D = 128
SEQ = 9
OUT_CH = 128

NW = 32
CHUNK = 128

GROUPS = 5
NODES_G = N_NODES // GROUPS
FLAT_G = NODES_G * SEQ
CPW = -(-FLAT_G // (CHUNK * NW))
FLAT_G_PAD = NW * CPW * CHUNK

M_BLK = 400
N_MBLKS = NODES_G // M_BLK

D_HALF = D // 2


def _sc_gather_body(x_hbm, idx_hbm, out_hbm, idx_v, buf_a, buf_b, gs_a, gs_b, ws_a, ws_b):
    wid = lax.axis_index("s") * 2 + lax.axis_index("c")
    base_c = wid * CPW
    pltpu.sync_copy(idx_hbm.at[wid], idx_v)

    bufs, gsems, wsems = (buf_a, buf_b), (gs_a, gs_b), (ws_a, ws_b)

    def g_start(c, b):
        pltpu.async_copy(x_hbm.at[idx_v.at[c]], bufs[b], gsems[b])

    def g_wait(c, b):
        pltpu.make_async_copy(x_hbm.at[idx_v.at[c]], bufs[b], gsems[b]).wait()

    def out_slice(c):
        return out_hbm.at[pl.ds((base_c + c) * CHUNK, CHUNK)]

    def w_start(c, b):
        pltpu.async_copy(bufs[b], out_slice(c), wsems[b])

    def w_wait(c, b):
        pltpu.make_async_copy(bufs[b], out_slice(c), wsems[b]).wait()

    g_start(0, 0)
    g_wait(0, 0)
    w_start(0, 0)
    g_start(1, 1)

    def step(i, carry):
        c1 = 1 + 2 * i
        g_wait(c1, 1)
        w_start(c1, 1)
        w_wait(c1 - 1, 0)
        g_start(c1 + 1, 0)
        c2 = c1 + 1
        g_wait(c2, 0)
        w_start(c2, 0)
        w_wait(c2 - 1, 1)
        g_start(c2 + 1, 1)
        return carry

    lax.fori_loop(0, (CPW - 2) // 2, step, 0)
    c = CPW - 1
    g_wait(c, 1)
    w_start(c, 1)
    w_wait(c - 1, 0)
    w_wait(c, 1)


def _sc_gather(x, idx3d):
    mesh = plsc.VectorSubcoreMesh(core_axis_name="c", subcore_axis_name="s")
    k = functools.partial(
        pl.kernel,
        mesh=mesh,
        compiler_params=pltpu.CompilerParams(use_tc_tiling_on_sc=False),
        out_type=jax.ShapeDtypeStruct((FLAT_G_PAD, D_HALF), jnp.float32),
        scratch_types=[
            pltpu.VMEM((CPW, CHUNK), jnp.int32),
            pltpu.VMEM((CHUNK, D_HALF), jnp.float32),
            pltpu.VMEM((CHUNK, D_HALF), jnp.float32),
            pltpu.SemaphoreType.DMA,
            pltpu.SemaphoreType.DMA,
            pltpu.SemaphoreType.DMA,
            pltpu.SemaphoreType.DMA,
        ],
    )(_sc_gather_body)
    return k(x, idx3d)


def _mm_body(*refs):
    g_refs, w_ref, b_ref, o_ref = refs[:SEQ], refs[SEQ], refs[SEQ + 1], refs[SEQ + 2]
    acc = jnp.broadcast_to(b_ref[...], (M_BLK, OUT_CH))
    for s in range(SEQ):
        acc = acc + jnp.dot(g_refs[s][...], w_ref[s],
                            preferred_element_type=jnp.float32)
    o_ref[...] = acc


def _tc_matmul(gathered, w3, b2):
    in_specs = [
        pl.BlockSpec((M_BLK, D), functools.partial(lambda i, s: (s * N_MBLKS + i, 0), s=s))
        for s in range(SEQ)
    ]
    in_specs.append(pl.BlockSpec((SEQ, D, OUT_CH), lambda i: (0, 0, 0)))
    in_specs.append(pl.BlockSpec((1, OUT_CH), lambda i: (0, 0)))
    return pl.pallas_call(
        _mm_body,
        grid=(N_MBLKS,),
        in_specs=in_specs,
        out_specs=pl.BlockSpec((M_BLK, OUT_CH), lambda i: (i, 0)),
        out_shape=jax.ShapeDtypeStruct((NODES_G, OUT_CH), jnp.float32),
    )(*([gathered] * SEQ), w3, b2)


def kernel(x, indices, W, b):
    x_bf = x.astype(jnp.bfloat16)
    x_v = lax.bitcast_convert_type(x_bf.reshape(N_NODES, D_HALF, 2), jnp.float32)
    w3 = W.reshape(SEQ, D, OUT_CH).astype(jnp.bfloat16)
    b2 = b.reshape(1, OUT_CH)
    outs = []
    for g in range(GROUPS):
        idx_g = indices[g * NODES_G:(g + 1) * NODES_G].astype(jnp.int32).T.reshape(-1)
        idx_g = jnp.pad(idx_g, (0, FLAT_G_PAD - FLAT_G)).reshape(NW, CPW, CHUNK)
        gathered = _sc_gather(x_v, idx_g)
        g_bf = lax.bitcast_convert_type(gathered, jnp.bfloat16).reshape(FLAT_G_PAD, D)
        outs.append(_tc_matmul(g_bf, w3, b2))
    return jnp.concatenate(outs, axis=0)


# speedup vs baseline: 5.1830x; 5.1830x over previous
import functools

import jax
import jax.numpy as jnp
from jax import lax
from jax.experimental import pallas as pl
from jax.experimental.pallas import tpu as pltpu
from jax.experimental.pallas import tpu_sc as plsc

N_NODES = 50000
D = 128
SEQ = 9
OUT_CH = 128

NW = 32
CHUNK = 128

GROUPS = 5
NODES_G = N_NODES // GROUPS
FLAT_G = NODES_G * SEQ
CPW = -(-FLAT_G // (CHUNK * NW))
FLAT_G_PAD = NW * CPW * CHUNK

M_BLK = 400
N_MBLKS = NODES_G // M_BLK

LEAD = 3


def _sc_gather_body(x_hbm, idx_hbm, out_hbm, idx_v, buf_a, buf_b, buf_c,
                    gs_a, gs_b, gs_c):
    wid = lax.axis_index("s") * 2 + lax.axis_index("c")
    base_c = wid * CPW
    pltpu.sync_copy(idx_hbm.at[wid], idx_v)

    bufs, gsems = (buf_a, buf_b, buf_c), (gs_a, gs_b, gs_c)

    def g_start(c, b):
        pltpu.async_copy(x_hbm.at[idx_v.at[c]], bufs[b], gsems[b])

    def g_wait(c, b):
        pltpu.make_async_copy(x_hbm.at[idx_v.at[c]], bufs[b], gsems[b]).wait()

    def w_sync(c, b):
        pltpu.sync_copy(bufs[b], out_hbm.at[pl.ds((base_c + c) * CHUNK, CHUNK)])

    for b in range(LEAD):
        g_start(b, b)

    n_main = (CPW - LEAD) // LEAD * LEAD

    def step(i, carry):
        c0 = i * LEAD
        for b in range(LEAD):
            g_wait(c0 + b, b)
            w_sync(c0 + b, b)
            g_start(c0 + b + LEAD, b)
        return carry

    lax.fori_loop(0, n_main // LEAD, step, 0)

    for c in range(n_main, CPW):
        b = c % LEAD
        g_wait(c, b)
        w_sync(c, b)
        if c + LEAD < CPW:
            g_start(c + LEAD, b)


def _sc_gather(x, idx3d):
    mesh = plsc.VectorSubcoreMesh(core_axis_name="c", subcore_axis_name="s")
    k = functools.partial(
        pl.kernel,
        mesh=mesh,
        out_type=jax.ShapeDtypeStruct((FLAT_G_PAD, D), jnp.float32),
        scratch_types=[
            pltpu.VMEM((CPW, CHUNK), jnp.int32),
            pltpu.VMEM((CHUNK, D), jnp.float32),
            pltpu.VMEM((CHUNK, D), jnp.float32),
            pltpu.VMEM((CHUNK, D), jnp.float32),
            pltpu.SemaphoreType.DMA,
            pltpu.SemaphoreType.DMA,
            pltpu.SemaphoreType.DMA,
        ],
    )(_sc_gather_body)
    return k(x, idx3d)


def _mm_body(*refs):
    g_refs, w_ref, b_ref, o_ref = refs[:SEQ], refs[SEQ], refs[SEQ + 1], refs[SEQ + 2]
    acc = jnp.broadcast_to(b_ref[...], (M_BLK, OUT_CH))
    for s in range(SEQ):
        acc = acc + jnp.dot(g_refs[s][...], w_ref[s],
                            preferred_element_type=jnp.float32)
    o_ref[...] = acc


def _tc_matmul(gathered, w3, b2):
    in_specs = [
        pl.BlockSpec((M_BLK, D), functools.partial(lambda i, s: (s * N_MBLKS + i, 0), s=s))
        for s in range(SEQ)
    ]
    in_specs.append(pl.BlockSpec((SEQ, D, OUT_CH), lambda i: (0, 0, 0)))
    in_specs.append(pl.BlockSpec((1, OUT_CH), lambda i: (0, 0)))
    return pl.pallas_call(
        _mm_body,
        grid=(N_MBLKS,),
        in_specs=in_specs,
        out_specs=pl.BlockSpec((M_BLK, OUT_CH), lambda i: (i, 0)),
        out_shape=jax.ShapeDtypeStruct((NODES_G, OUT_CH), jnp.float32),
    )(*([gathered] * SEQ), w3, b2)


def kernel(x, indices, W, b):
    w3 = W.reshape(SEQ, D, OUT_CH)
    b2 = b.reshape(1, OUT_CH)
    outs = []
    for g in range(GROUPS):
        idx_g = indices[g * NODES_G:(g + 1) * NODES_G].astype(jnp.int32).T.reshape(-1)
        idx_g = jnp.pad(idx_g, (0, FLAT_G_PAD - FLAT_G)).reshape(NW, CPW, CHUNK)
        gathered = _sc_gather(x, idx_g)
        outs.append(_tc_matmul(gathered, w3, b2))
    return jnp.concatenate(outs, axis=0)


# speedup vs baseline: 5.4160x; 1.0450x over previous
import functools

import jax
import jax.numpy as jnp
from jax import lax
from jax.experimental import pallas as pl
from jax.experimental.pallas import tpu as pltpu
from jax.experimental.pallas import tpu_sc as plsc

N_NODES = 50000
D = 128
SEQ = 9
OUT_CH = 128

NW = 32
CHUNK = 128

GROUPS = 5
NODES_G = N_NODES // GROUPS
FLAT_G = NODES_G * SEQ
CPW = -(-FLAT_G // (CHUNK * NW))
FLAT_G_PAD = NW * CPW * CHUNK

M_BLK = 400
N_MBLKS = NODES_G // M_BLK

LEAD = 3


def _sc_gather_body(x_hbm, idx_hbm, out_hbm, idx_v, buf_a, buf_b, buf_c,
                    gs_a, gs_b, gs_c):
    wid = lax.axis_index("s") * 2 + lax.axis_index("c")
    base_c = wid * CPW
    pltpu.sync_copy(idx_hbm.at[wid], idx_v)

    bufs, gsems = (buf_a, buf_b, buf_c), (gs_a, gs_b, gs_c)

    def g_start(c, b):
        pltpu.async_copy(x_hbm.at[idx_v.at[c]], bufs[b], gsems[b])

    def g_wait(c, b):
        pltpu.make_async_copy(x_hbm.at[idx_v.at[c]], bufs[b], gsems[b]).wait()

    def w_sync(c, b):
        pltpu.sync_copy(bufs[b], out_hbm.at[pl.ds((base_c + c) * CHUNK, CHUNK)])

    for b in range(LEAD):
        g_start(b, b)

    n_main = (CPW - LEAD) // LEAD * LEAD

    def step(i, carry):
        c0 = i * LEAD
        for b in range(LEAD):
            g_wait(c0 + b, b)
            w_sync(c0 + b, b)
            g_start(c0 + b + LEAD, b)
        return carry

    lax.fori_loop(0, n_main // LEAD, step, 0)

    for c in range(n_main, CPW):
        b = c % LEAD
        g_wait(c, b)
        w_sync(c, b)
        if c + LEAD < CPW:
            g_start(c + LEAD, b)


def _sc_gather(x, idx3d):
    mesh = plsc.VectorSubcoreMesh(core_axis_name="c", subcore_axis_name="s")
    k = functools.partial(
        pl.kernel,
        mesh=mesh,
        out_type=jax.ShapeDtypeStruct((FLAT_G_PAD, D), jnp.float32),
        scratch_types=[
            pltpu.VMEM((CPW, CHUNK), jnp.int32),
            pltpu.VMEM((CHUNK, D), jnp.float32),
            pltpu.VMEM((CHUNK, D), jnp.float32),
            pltpu.VMEM((CHUNK, D), jnp.float32),
            pltpu.SemaphoreType.DMA,
            pltpu.SemaphoreType.DMA,
            pltpu.SemaphoreType.DMA,
        ],
    )(_sc_gather_body)
    return k(x, idx3d)


def _mm_body(*refs):
    g_refs, w_ref, b_ref = refs[:SEQ], refs[SEQ], refs[SEQ + 1]
    o_ref = refs[-1]
    acc = jnp.broadcast_to(b_ref[...], (M_BLK, OUT_CH))
    for s in range(SEQ):
        acc = acc + jnp.dot(g_refs[s][...], w_ref[s],
                            preferred_element_type=jnp.float32)
    o_ref[...] = acc


def _tc_matmul(gathered, w3, b2, base_blk, out_prev):
    in_specs = [
        pl.BlockSpec((M_BLK, D), functools.partial(lambda i, s: (s * N_MBLKS + i, 0), s=s))
        for s in range(SEQ)
    ]
    in_specs.append(pl.BlockSpec((SEQ, D, OUT_CH), lambda i: (0, 0, 0)))
    in_specs.append(pl.BlockSpec((1, OUT_CH), lambda i: (0, 0)))
    args = [*([gathered] * SEQ), w3, b2]
    aliases = {}
    if out_prev is not None:
        in_specs.append(pl.BlockSpec(memory_space=pl.ANY))
        args.append(out_prev)
        aliases = {SEQ + 2: 0}
    return pl.pallas_call(
        _mm_body,
        grid=(N_MBLKS,),
        in_specs=in_specs,
        out_specs=pl.BlockSpec((M_BLK, OUT_CH),
                               functools.partial(lambda i, bb: (bb + i, 0), bb=base_blk)),
        out_shape=jax.ShapeDtypeStruct((N_NODES, OUT_CH), jnp.float32),
        input_output_aliases=aliases,
    )(*args)


def kernel(x, indices, W, b):
    w3 = W.reshape(SEQ, D, OUT_CH)
    b2 = b.reshape(1, OUT_CH)
    out = None
    for g in range(GROUPS):
        idx_g = indices[g * NODES_G:(g + 1) * NODES_G].astype(jnp.int32).T.reshape(-1)
        idx_g = jnp.pad(idx_g, (0, FLAT_G_PAD - FLAT_G)).reshape(NW, CPW, CHUNK)
        gathered = _sc_gather(x, idx_g)
        out = _tc_matmul(gathered, w3, b2, g * N_MBLKS, out)
    return out
